# baseline (speedup 1.0000x reference)
import functools

import jax
import jax.numpy as jnp
from jax import lax
from jax.experimental import pallas as pl
from jax.experimental.pallas import tpu as pltpu
from jax.experimental.pallas import tpu_sc as plsc

S = 4096
DIM = 2048
K = 2048
SB = 512


def _topk_body(scol_ref, srow_ref, idx_ref):
    scol = scol_ref[...]
    srow = srow_ref[...]
    nb = S // SB
    r0 = lax.broadcasted_iota(jnp.int32, (SB, SB), 0)
    c0 = lax.broadcasted_iota(jnp.int32, (SB, SB), 1)
    eye = r0 == c0
    rc = [None] * nb
    rr = [None] * nb
    for a in range(nb):
        sa_col = scol[a * SB:(a + 1) * SB, :]
        sa_row = srow[:, a * SB:(a + 1) * SB]
        rel = (sa_row > sa_col) | ((sa_row == sa_col) & (c0 < r0))
        rc[a] = jnp.sum(jnp.where(rel, 1.0, 0.0), axis=1, keepdims=True)
        for b in range(a + 1, nb):
            sb_row = srow[:, b * SB:(b + 1) * SB]
            gt = jnp.where(sb_row > sa_col, 1.0, 0.0)
            rc[a] = rc[a] + jnp.sum(gt, axis=1, keepdims=True)
            cs = jnp.sum(gt, axis=0, keepdims=True)
            rr[b] = (SB - cs) if rr[b] is None else rr[b] + (SB - cs)
    rrow = jnp.concatenate(
        [jnp.sum(jnp.where(eye, rc[b], 0.0), axis=0, keepdims=True)
         + (rr[b] if rr[b] is not None else 0.0) for b in range(nb)], axis=1)
    tok = lax.broadcasted_iota(jnp.int32, (SB, S), 1).astype(jnp.float32)
    for ob in range(K // SB):
        rcol = (lax.broadcasted_iota(jnp.int32, (SB, S), 0)
                + ob * SB).astype(jnp.float32)
        hit = rrow == rcol
        idxb = jnp.sum(jnp.where(hit, tok, 0.0), axis=1, keepdims=True)
        idx_ref[pl.ds(ob * SB, SB), :] = idxb.astype(jnp.int32)


def _topk(s_col, s_row):
    return pl.pallas_call(
        _topk_body,
        in_specs=[
            pl.BlockSpec((S, 1), lambda: (0, 0)),
            pl.BlockSpec((1, S), lambda: (0, 0)),
        ],
        out_specs=pl.BlockSpec((K, 1), lambda: (0, 0)),
        out_shape=jax.ShapeDtypeStruct((K, 1), jnp.int32),
    )(s_col, s_row)


_GCHUNK = 16
_NCHUNK = 4


def _gather_sc(x2d, idx_flat):
    nw = 32
    rows_per_w = K // nw

    @functools.partial(
        pl.kernel,
        mesh=plsc.VectorSubcoreMesh(core_axis_name="c", subcore_axis_name="s"),
        out_type=jax.ShapeDtypeStruct((K, DIM), jnp.float32),
        scratch_types=[
            pltpu.VMEM((rows_per_w,), jnp.int32),
            pltpu.VMEM((_GCHUNK, DIM), jnp.float32),
            pltpu.VMEM((_GCHUNK, DIM), jnp.float32),
            pltpu.SemaphoreType.DMA,
            pltpu.SemaphoreType.DMA,
        ],
    )
    def gather_k(x_hbm, idx_hbm, out_hbm, idx_v, rows_a, rows_b, sem_a, sem_b):
        wid = lax.axis_index("s") * 2 + lax.axis_index("c")
        base = wid * rows_per_w
        pltpu.sync_copy(idx_hbm.at[pl.ds(base, rows_per_w)], idx_v)
        bufs = (rows_a, rows_b)
        sems = (sem_a, sem_b)
        copies = [
            pltpu.make_async_copy(
                x_hbm.at[idx_v[pl.ds(c * _GCHUNK, _GCHUNK)]],
                bufs[c % 2], sems[c % 2])
            for c in range(_NCHUNK)
        ]
        copies[0].start()
        for c in range(_NCHUNK):
            if c + 1 < _NCHUNK:
                copies[c + 1].start()
            copies[c].wait()
            pltpu.sync_copy(bufs[c % 2],
                            out_hbm.at[pl.ds(base + c * _GCHUNK, _GCHUNK)])

    return gather_k(x2d, idx_flat)


def _step_body(sel_ref, q1_ref, bq1_ref, q2_ref, bq2_ref, step_ref,
               sum_ref, amax_ref):
    i = pl.program_id(0)
    nb = K // SB
    blk = sel_ref[...]
    psum = jnp.sum(blk, axis=0, keepdims=True)
    pmax = jnp.max(jnp.abs(blk), axis=0, keepdims=True)

    @pl.when(i == 0)
    def _():
        sum_ref[...] = psum
        amax_ref[...] = pmax

    @pl.when(i > 0)
    def _():
        sum_ref[...] = sum_ref[...] + psum
        amax_ref[...] = jnp.maximum(amax_ref[...], pmax)

    @pl.when(i == nb - 1)
    def _():
        m = sum_ref[...] * jnp.float32(1.0 / K)
        t1 = jnp.dot(m, q1_ref[...], preferred_element_type=jnp.float32)
        t1 = jnp.maximum(t1 + bq1_ref[...], 0.0)
        logits = jnp.dot(t1, q2_ref[...],
                         preferred_element_type=jnp.float32) + bq2_ref[...]
        e = jnp.exp(logits - jnp.max(logits, axis=1, keepdims=True))
        p = e / jnp.sum(e, axis=1, keepdims=True)
        lane = lax.broadcasted_iota(jnp.int32, (1, 8), 1)
        pm = jnp.max(p, axis=1, keepdims=True)
        amx = jnp.min(jnp.where(p == pm, lane, 8), axis=1, keepdims=True)
        bits = amx + 1
        pow2 = jnp.left_shift(1, bits).astype(jnp.float32)
        step_ref[...] = amax_ref[...] / pow2[0, 0]


def _quant_step(sel, Q1, bq1r, Q2, bq2r):
    return pl.pallas_call(
        _step_body,
        grid=(K // SB,),
        in_specs=[
            pl.BlockSpec((SB, DIM), lambda i: (i, 0)),
            pl.BlockSpec((DIM, DIM // 4), lambda i: (0, 0)),
            pl.BlockSpec((1, DIM // 4), lambda i: (0, 0)),
            pl.BlockSpec((DIM // 4, 8), lambda i: (0, 0)),
            pl.BlockSpec((1, 8), lambda i: (0, 0)),
        ],
        out_specs=pl.BlockSpec((1, DIM), lambda i: (0, 0)),
        out_shape=jax.ShapeDtypeStruct((1, DIM), jnp.float32),
        scratch_shapes=[
            pltpu.VMEM((1, DIM), jnp.float32),
            pltpu.VMEM((1, DIM), jnp.float32),
        ],
    )(sel, Q1, bq1r, Q2, bq2r)


def _quantize_body(sel_ref, step_ref, out_ref):
    q = sel_ref[...] / step_ref[...]
    out_ref[...] = jnp.round(q) * step_ref[...]


def _quantize(sel, step_row):
    return pl.pallas_call(
        _quantize_body,
        grid=(K // SB,),
        in_specs=[
            pl.BlockSpec((SB, DIM), lambda i: (i, 0)),
            pl.BlockSpec((1, DIM), lambda i: (0, 0)),
        ],
        out_specs=pl.BlockSpec((SB, DIM), lambda i: (i, 0)),
        out_shape=jax.ShapeDtypeStruct((K, DIM), jnp.float32),
    )(sel, step_row)


def kernel(x, W1, b1, ln_g, ln_b, W2, b2, Q1, bq1, Q2, bq2):
    x2d = x.reshape(S, DIM)
    t = x2d @ W1 + b1
    mu = jnp.mean(t, axis=-1, keepdims=True)
    var = jnp.mean((t - mu) ** 2, axis=-1, keepdims=True)
    h = jax.nn.relu((t - mu) / jnp.sqrt(var + 1e-5) * ln_g + ln_b)
    s = jax.nn.sigmoid(h @ W2 + b2)
    idx_col = _topk(s, s.reshape(1, S))
    sel = _gather_sc(x2d, idx_col.reshape(K))
    step_row = _quant_step(sel, Q1, bq1.reshape(1, -1), Q2, bq2.reshape(1, -1))
    out = _quantize(sel, step_row)
    return out.reshape(1, K, DIM)


# speedup vs baseline: 1.0061x; 1.0061x over previous
import functools

import jax
import jax.numpy as jnp
from jax import lax
from jax.experimental import pallas as pl
from jax.experimental.pallas import tpu as pltpu
from jax.experimental.pallas import tpu_sc as plsc

S = 4096
DIM = 2048
K = 2048
SB = 512


def _topk_body(scol_ref, srow_ref, idx_ref):
    scol = scol_ref[...]
    srow = srow_ref[...]
    nb = S // SB
    r0 = lax.broadcasted_iota(jnp.int32, (SB, SB), 0)
    c0 = lax.broadcasted_iota(jnp.int32, (SB, SB), 1)
    eye = r0 == c0
    rc = [None] * nb
    rr = [None] * nb
    for a in range(nb):
        sa_col = scol[a * SB:(a + 1) * SB, :]
        sa_row = srow[:, a * SB:(a + 1) * SB]
        rel = (sa_row > sa_col) | ((sa_row == sa_col) & (c0 < r0))
        rc[a] = jnp.sum(jnp.where(rel, 1.0, 0.0), axis=1, keepdims=True)
        for b in range(a + 1, nb):
            sb_row = srow[:, b * SB:(b + 1) * SB]
            gt = jnp.where(sb_row > sa_col, 1.0, 0.0)
            rc[a] = rc[a] + jnp.sum(gt, axis=1, keepdims=True)
            cs = jnp.sum(gt, axis=0, keepdims=True)
            rr[b] = (SB - cs) if rr[b] is None else rr[b] + (SB - cs)
    rrow = jnp.concatenate(
        [jnp.sum(jnp.where(eye, rc[b], 0.0), axis=0, keepdims=True)
         + (rr[b] if rr[b] is not None else 0.0) for b in range(nb)], axis=1)
    tok = lax.broadcasted_iota(jnp.int32, (SB, S), 1).astype(jnp.float32)
    for ob in range(K // SB):
        rcol = (lax.broadcasted_iota(jnp.int32, (SB, S), 0)
                + ob * SB).astype(jnp.float32)
        hit = rrow == rcol
        idxb = jnp.sum(jnp.where(hit, tok, 0.0), axis=1, keepdims=True)
        idx_ref[pl.ds(ob * SB, SB), :] = idxb.astype(jnp.int32)


def _topk(s_col, s_row):
    return pl.pallas_call(
        _topk_body,
        in_specs=[
            pl.BlockSpec((S, 1), lambda: (0, 0)),
            pl.BlockSpec((1, S), lambda: (0, 0)),
        ],
        out_specs=pl.BlockSpec((K, 1), lambda: (0, 0)),
        out_shape=jax.ShapeDtypeStruct((K, 1), jnp.int32),
    )(s_col, s_row)


_GCHUNK = 32


def _gather_sc(x2d, idx_flat):
    nw = 32
    rows_per_w = K // nw

    @functools.partial(
        pl.kernel,
        mesh=plsc.VectorSubcoreMesh(core_axis_name="c", subcore_axis_name="s"),
        out_type=jax.ShapeDtypeStruct((K, DIM), jnp.float32),
        scratch_types=[
            pltpu.VMEM((_GCHUNK,), jnp.int32),
            pltpu.VMEM((_GCHUNK, DIM), jnp.float32),
            pltpu.SemaphoreType.DMA,
        ],
    )
    def gather_k(x_hbm, idx_hbm, out_hbm, idx_v, rows_v, sem):
        wid = lax.axis_index("s") * 2 + lax.axis_index("c")
        for c in range(rows_per_w // _GCHUNK):
            base = wid * rows_per_w + c * _GCHUNK
            pltpu.sync_copy(idx_hbm.at[pl.ds(base, _GCHUNK)], idx_v)
            pltpu.async_copy(x_hbm.at[idx_v], rows_v, sem).wait()
            pltpu.sync_copy(rows_v, out_hbm.at[pl.ds(base, _GCHUNK)])

    return gather_k(x2d, idx_flat)


def _step_body(sel_ref, q1_ref, bq1_ref, q2_ref, bq2_ref, step_ref,
               sum_ref, amax_ref):
    i = pl.program_id(0)
    nb = K // SB
    blk = sel_ref[...]
    psum = jnp.sum(blk, axis=0, keepdims=True)
    pmax = jnp.max(jnp.abs(blk), axis=0, keepdims=True)

    @pl.when(i == 0)
    def _():
        sum_ref[...] = psum
        amax_ref[...] = pmax

    @pl.when(i > 0)
    def _():
        sum_ref[...] = sum_ref[...] + psum
        amax_ref[...] = jnp.maximum(amax_ref[...], pmax)

    @pl.when(i == nb - 1)
    def _():
        m = sum_ref[...] * jnp.float32(1.0 / K)
        t1 = jnp.dot(m, q1_ref[...], preferred_element_type=jnp.float32)
        t1 = jnp.maximum(t1 + bq1_ref[...], 0.0)
        logits = jnp.dot(t1, q2_ref[...],
                         preferred_element_type=jnp.float32) + bq2_ref[...]
        e = jnp.exp(logits - jnp.max(logits, axis=1, keepdims=True))
        p = e / jnp.sum(e, axis=1, keepdims=True)
        lane = lax.broadcasted_iota(jnp.int32, (1, 8), 1)
        pm = jnp.max(p, axis=1, keepdims=True)
        amx = jnp.min(jnp.where(p == pm, lane, 8), axis=1, keepdims=True)
        bits = amx + 1
        pow2 = jnp.left_shift(1, bits).astype(jnp.float32)
        step_ref[...] = amax_ref[...] / pow2[0, 0]


def _quant_step(sel, Q1, bq1r, Q2, bq2r):
    return pl.pallas_call(
        _step_body,
        grid=(K // SB,),
        in_specs=[
            pl.BlockSpec((SB, DIM), lambda i: (i, 0)),
            pl.BlockSpec((DIM, DIM // 4), lambda i: (0, 0)),
            pl.BlockSpec((1, DIM // 4), lambda i: (0, 0)),
            pl.BlockSpec((DIM // 4, 8), lambda i: (0, 0)),
            pl.BlockSpec((1, 8), lambda i: (0, 0)),
        ],
        out_specs=pl.BlockSpec((1, DIM), lambda i: (0, 0)),
        out_shape=jax.ShapeDtypeStruct((1, DIM), jnp.float32),
        scratch_shapes=[
            pltpu.VMEM((1, DIM), jnp.float32),
            pltpu.VMEM((1, DIM), jnp.float32),
        ],
    )(sel, Q1, bq1r, Q2, bq2r)


def _quantize_body(sel_ref, step_ref, out_ref):
    q = sel_ref[...] / step_ref[...]
    out_ref[...] = jnp.round(q) * step_ref[...]


def _quantize(sel, step_row):
    return pl.pallas_call(
        _quantize_body,
        grid=(K // SB,),
        in_specs=[
            pl.BlockSpec((SB, DIM), lambda i: (i, 0)),
            pl.BlockSpec((1, DIM), lambda i: (0, 0)),
        ],
        out_specs=pl.BlockSpec((SB, DIM), lambda i: (i, 0)),
        out_shape=jax.ShapeDtypeStruct((K, DIM), jnp.float32),
    )(sel, step_row)


def kernel(x, W1, b1, ln_g, ln_b, W2, b2, Q1, bq1, Q2, bq2):
    x2d = x.reshape(S, DIM)
    t = x2d @ W1 + b1
    mu = jnp.mean(t, axis=-1, keepdims=True)
    var = jnp.mean((t - mu) ** 2, axis=-1, keepdims=True)
    h = jax.nn.relu((t - mu) / jnp.sqrt(var + 1e-5) * ln_g + ln_b)
    s = jax.nn.sigmoid(h @ W2 + b2)
    idx_col = _topk(s, s.reshape(1, S))
    sel = _gather_sc(x2d, idx_col.reshape(K))
    step_row = _quant_step(sel, Q1, bq1.reshape(1, -1), Q2, bq2.reshape(1, -1))
    out = _quantize(sel, step_row)
    return out.reshape(1, K, DIM)


# speedup vs baseline: 1.0361x; 1.0298x over previous
import functools

import jax
import jax.numpy as jnp
from jax import lax
from jax.experimental import pallas as pl
from jax.experimental.pallas import tpu as pltpu
from jax.experimental.pallas import tpu_sc as plsc

S = 4096
DIM = 2048
K = 2048
SB = 512
QB = 256


def _topk_body(scol_ref, srow_ref, idx_ref):
    scol = scol_ref[...]
    srow = srow_ref[...]
    nb = S // SB
    r0 = lax.broadcasted_iota(jnp.int32, (SB, SB), 0)
    c0 = lax.broadcasted_iota(jnp.int32, (SB, SB), 1)
    eye = r0 == c0
    rc = [None] * nb
    rr = [None] * nb
    for a in range(nb):
        sa_col = scol[a * SB:(a + 1) * SB, :]
        sa_row = srow[:, a * SB:(a + 1) * SB]
        rel = (sa_row > sa_col) | ((sa_row == sa_col) & (c0 < r0))
        rc[a] = jnp.sum(jnp.where(rel, 1.0, 0.0), axis=1, keepdims=True)
        for b in range(a + 1, nb):
            sb_row = srow[:, b * SB:(b + 1) * SB]
            gt = jnp.where(sb_row > sa_col, 1.0, 0.0)
            rc[a] = rc[a] + jnp.sum(gt, axis=1, keepdims=True)
            cs = jnp.sum(gt, axis=0, keepdims=True)
            rr[b] = (SB - cs) if rr[b] is None else rr[b] + (SB - cs)
    rrow = jnp.concatenate(
        [jnp.sum(jnp.where(eye, rc[b], 0.0), axis=0, keepdims=True)
         + (rr[b] if rr[b] is not None else 0.0) for b in range(nb)], axis=1)
    tok = lax.broadcasted_iota(jnp.int32, (SB, S), 1).astype(jnp.float32)
    for ob in range(K // SB):
        rcol = (lax.broadcasted_iota(jnp.int32, (SB, S), 0)
                + ob * SB).astype(jnp.float32)
        hit = rrow == rcol
        idxb = jnp.sum(jnp.where(hit, tok, 0.0), axis=1, keepdims=True)
        idx_ref[pl.ds(ob * SB, SB), :] = idxb.astype(jnp.int32)


def _topk(s_col, s_row):
    return pl.pallas_call(
        _topk_body,
        in_specs=[
            pl.BlockSpec((S, 1), lambda: (0, 0)),
            pl.BlockSpec((1, S), lambda: (0, 0)),
        ],
        out_specs=pl.BlockSpec((K, 1), lambda: (0, 0)),
        out_shape=jax.ShapeDtypeStruct((K, 1), jnp.int32),
    )(s_col, s_row)


_GCHUNK = 32


def _gather_sc(x2d, idx_flat):
    nw = 32
    rows_per_w = K // nw

    @functools.partial(
        pl.kernel,
        mesh=plsc.VectorSubcoreMesh(core_axis_name="c", subcore_axis_name="s"),
        out_type=jax.ShapeDtypeStruct((K, DIM), jnp.float32),
        scratch_types=[
            pltpu.VMEM((_GCHUNK,), jnp.int32),
            pltpu.VMEM((_GCHUNK, DIM), jnp.float32),
            pltpu.SemaphoreType.DMA,
        ],
    )
    def gather_k(x_hbm, idx_hbm, out_hbm, idx_v, rows_v, sem):
        wid = lax.axis_index("s") * 2 + lax.axis_index("c")
        for c in range(rows_per_w // _GCHUNK):
            base = wid * rows_per_w + c * _GCHUNK
            pltpu.sync_copy(idx_hbm.at[pl.ds(base, _GCHUNK)], idx_v)
            pltpu.async_copy(x_hbm.at[idx_v], rows_v, sem).wait()
            pltpu.sync_copy(rows_v, out_hbm.at[pl.ds(base, _GCHUNK)])

    return gather_k(x2d, idx_flat)


def _quant_fused_body(sel_ref, q1_ref, bq1_ref, q2_ref, bq2_ref, out_ref,
                      selbuf, sum_ref, amax_ref):
    i = pl.program_id(0)
    nb = K // QB
    blk = sel_ref[...]
    selbuf[pl.ds(i * QB, QB), :] = blk
    psum = jnp.sum(blk, axis=0, keepdims=True)
    pmax = jnp.max(jnp.abs(blk), axis=0, keepdims=True)

    @pl.when(i == 0)
    def _():
        sum_ref[...] = psum
        amax_ref[...] = pmax

    @pl.when(i > 0)
    def _():
        sum_ref[...] = sum_ref[...] + psum
        amax_ref[...] = jnp.maximum(amax_ref[...], pmax)

    @pl.when(i == nb - 1)
    def _():
        m = sum_ref[...] * jnp.float32(1.0 / K)
        t1 = jnp.dot(m, q1_ref[...], preferred_element_type=jnp.float32)
        t1 = jnp.maximum(t1 + bq1_ref[...], 0.0)
        logits = jnp.dot(t1, q2_ref[...],
                         preferred_element_type=jnp.float32) + bq2_ref[...]
        e = jnp.exp(logits - jnp.max(logits, axis=1, keepdims=True))
        p = e / jnp.sum(e, axis=1, keepdims=True)
        lane = lax.broadcasted_iota(jnp.int32, (1, 8), 1)
        pm = jnp.max(p, axis=1, keepdims=True)
        amx = jnp.min(jnp.where(p == pm, lane, 8), axis=1, keepdims=True)
        bits = amx + 1
        pow2 = jnp.left_shift(1, bits).astype(jnp.float32)
        step = amax_ref[...] / pow2[0, 0]
        q = selbuf[...] / step
        out_ref[...] = jnp.round(q) * step


def _quant_fused(sel, Q1, bq1r, Q2, bq2r):
    return pl.pallas_call(
        _quant_fused_body,
        grid=(K // QB,),
        in_specs=[
            pl.BlockSpec((QB, DIM), lambda i: (i, 0)),
            pl.BlockSpec((DIM, DIM // 4), lambda i: (0, 0)),
            pl.BlockSpec((1, DIM // 4), lambda i: (0, 0)),
            pl.BlockSpec((DIM // 4, 8), lambda i: (0, 0)),
            pl.BlockSpec((1, 8), lambda i: (0, 0)),
        ],
        out_specs=pl.BlockSpec((K, DIM), lambda i: (0, 0)),
        out_shape=jax.ShapeDtypeStruct((K, DIM), jnp.float32),
        scratch_shapes=[
            pltpu.VMEM((K, DIM), jnp.float32),
            pltpu.VMEM((1, DIM), jnp.float32),
            pltpu.VMEM((1, DIM), jnp.float32),
        ],
    )(sel, Q1, bq1r, Q2, bq2r)


def kernel(x, W1, b1, ln_g, ln_b, W2, b2, Q1, bq1, Q2, bq2):
    x2d = x.reshape(S, DIM)
    t = x2d @ W1 + b1
    mu = jnp.mean(t, axis=-1, keepdims=True)
    var = jnp.mean((t - mu) ** 2, axis=-1, keepdims=True)
    h = jax.nn.relu((t - mu) / jnp.sqrt(var + 1e-5) * ln_g + ln_b)
    s = jax.nn.sigmoid(h @ W2 + b2)
    idx_col = _topk(s, s.reshape(1, S))
    sel = _gather_sc(x2d, idx_col.reshape(K))
    out = _quant_fused(sel, Q1, bq1.reshape(1, -1), Q2, bq2.reshape(1, -1))
    return out.reshape(1, K, DIM)


# speedup vs baseline: 1.0445x; 1.0081x over previous
import functools

import jax
import jax.numpy as jnp
from jax import lax
from jax.experimental import pallas as pl
from jax.experimental.pallas import tpu as pltpu
from jax.experimental.pallas import tpu_sc as plsc

S = 4096
DIM = 2048
K = 2048
SB = 512
QB = 256


def _topk_body(scol_ref, srow_ref, idx_ref):
    scol = scol_ref[...]
    srow = srow_ref[...]
    nb = S // SB
    r0 = lax.broadcasted_iota(jnp.int32, (SB, SB), 0)
    c0 = lax.broadcasted_iota(jnp.int32, (SB, SB), 1)
    eye = r0 == c0
    rc = [None] * nb
    rr = [None] * nb
    for a in range(nb):
        sa_col = scol[a * SB:(a + 1) * SB, :]
        sa_row = srow[:, a * SB:(a + 1) * SB]
        rel = (sa_row > sa_col) | ((sa_row == sa_col) & (c0 < r0))
        rc[a] = jnp.sum(jnp.where(rel, 1.0, 0.0), axis=1, keepdims=True)
        for b in range(a + 1, nb):
            sb_row = srow[:, b * SB:(b + 1) * SB]
            gt = jnp.where(sb_row > sa_col, 1.0, 0.0)
            rc[a] = rc[a] + jnp.sum(gt, axis=1, keepdims=True)
            cs = jnp.sum(gt, axis=0, keepdims=True)
            rr[b] = (SB - cs) if rr[b] is None else rr[b] + (SB - cs)
    rrow = jnp.concatenate(
        [jnp.sum(jnp.where(eye, rc[b], 0.0), axis=0, keepdims=True)
         + (rr[b] if rr[b] is not None else 0.0) for b in range(nb)], axis=1)
    tok = lax.broadcasted_iota(jnp.int32, (SB, S), 1).astype(jnp.float32)
    for ob in range(K // SB):
        rcol = (lax.broadcasted_iota(jnp.int32, (SB, S), 0)
                + ob * SB).astype(jnp.float32)
        hit = rrow == rcol
        idxb = jnp.sum(jnp.where(hit, tok, 0.0), axis=1, keepdims=True)
        idx_ref[pl.ds(ob * SB, SB), :] = idxb.astype(jnp.int32)


def _topk(s_col, s_row):
    return pl.pallas_call(
        _topk_body,
        in_specs=[
            pl.BlockSpec((S, 1), lambda: (0, 0)),
            pl.BlockSpec((1, S), lambda: (0, 0)),
        ],
        out_specs=pl.BlockSpec((K, 1), lambda: (0, 0)),
        out_shape=jax.ShapeDtypeStruct((K, 1), jnp.int32),
    )(s_col, s_row)


_GCHUNK = 32


def _gather_sc(x2d, idx_flat):
    nw = 32
    rows_per_w = K // nw

    @functools.partial(
        pl.kernel,
        mesh=plsc.VectorSubcoreMesh(core_axis_name="c", subcore_axis_name="s"),
        out_type=jax.ShapeDtypeStruct((K, DIM), jnp.float32),
        scratch_types=[
            pltpu.VMEM((_GCHUNK,), jnp.int32),
            pltpu.VMEM((_GCHUNK, DIM), jnp.float32),
            pltpu.SemaphoreType.DMA,
        ],
    )
    def gather_k(x_hbm, idx_hbm, out_hbm, idx_v, rows_v, sem):
        wid = lax.axis_index("s") * 2 + lax.axis_index("c")
        for c in range(rows_per_w // _GCHUNK):
            base = wid * rows_per_w + c * _GCHUNK
            pltpu.sync_copy(idx_hbm.at[pl.ds(base, _GCHUNK)], idx_v)
            pltpu.async_copy(x_hbm.at[idx_v], rows_v, sem).wait()
            pltpu.sync_copy(rows_v, out_hbm.at[pl.ds(base, _GCHUNK)])

    return gather_k(x2d, idx_flat)


def _quant_fused_body(sel_ref, q1_ref, bq1_ref, q2_ref, bq2_ref, out_ref,
                      selbuf, sum_ref, amax_ref):
    i = pl.program_id(0)
    nb = K // QB

    @pl.when(i < nb)
    def _():
        blk = sel_ref[...]
        selbuf[pl.ds(i * QB, QB), :] = blk
        psum = jnp.sum(blk, axis=0, keepdims=True)
        pmax = jnp.max(jnp.abs(blk), axis=0, keepdims=True)

        @pl.when(i == 0)
        def _():
            sum_ref[...] = psum
            amax_ref[...] = pmax

        @pl.when(i > 0)
        def _():
            sum_ref[...] = sum_ref[...] + psum
            amax_ref[...] = jnp.maximum(amax_ref[...], pmax)

    @pl.when(i == nb - 1)
    def _():
        m = sum_ref[...] * jnp.float32(1.0 / K)
        t1 = jnp.dot(m, q1_ref[...], preferred_element_type=jnp.float32)
        t1 = jnp.maximum(t1 + bq1_ref[...], 0.0)
        logits = jnp.dot(t1, q2_ref[...],
                         preferred_element_type=jnp.float32) + bq2_ref[...]
        e = jnp.exp(logits - jnp.max(logits, axis=1, keepdims=True))
        p = e / jnp.sum(e, axis=1, keepdims=True)
        lane = lax.broadcasted_iota(jnp.int32, (1, 8), 1)
        pm = jnp.max(p, axis=1, keepdims=True)
        amx = jnp.min(jnp.where(p == pm, lane, 8), axis=1, keepdims=True)
        bits = amx + 1
        pow2 = jnp.left_shift(1, bits).astype(jnp.float32)
        amax_ref[...] = amax_ref[...] / pow2[0, 0]

    @pl.when(i >= nb)
    def _():
        j = i - nb
        step = amax_ref[...]
        q = selbuf[pl.ds(j * QB, QB), :] / step
        out_ref[...] = jnp.round(q) * step


def _quant_fused(sel, Q1, bq1r, Q2, bq2r):
    nb = K // QB
    return pl.pallas_call(
        _quant_fused_body,
        grid=(2 * nb,),
        in_specs=[
            pl.BlockSpec((QB, DIM), lambda i: (jnp.minimum(i, nb - 1), 0)),
            pl.BlockSpec((DIM, DIM // 4), lambda i: (0, 0)),
            pl.BlockSpec((1, DIM // 4), lambda i: (0, 0)),
            pl.BlockSpec((DIM // 4, 8), lambda i: (0, 0)),
            pl.BlockSpec((1, 8), lambda i: (0, 0)),
        ],
        out_specs=pl.BlockSpec((QB, DIM),
                               lambda i: (jnp.maximum(i - nb, 0), 0)),
        out_shape=jax.ShapeDtypeStruct((K, DIM), jnp.float32),
        scratch_shapes=[
            pltpu.VMEM((K, DIM), jnp.float32),
            pltpu.VMEM((1, DIM), jnp.float32),
            pltpu.VMEM((1, DIM), jnp.float32),
        ],
    )(sel, Q1, bq1r, Q2, bq2r)


def kernel(x, W1, b1, ln_g, ln_b, W2, b2, Q1, bq1, Q2, bq2):
    x2d = x.reshape(S, DIM)
    t = x2d @ W1 + b1
    mu = jnp.mean(t, axis=-1, keepdims=True)
    var = jnp.mean((t - mu) ** 2, axis=-1, keepdims=True)
    h = jax.nn.relu((t - mu) / jnp.sqrt(var + 1e-5) * ln_g + ln_b)
    s = jax.nn.sigmoid(h @ W2 + b2)
    idx_col = _topk(s, s.reshape(1, S))
    sel = _gather_sc(x2d, idx_col.reshape(K))
    out = _quant_fused(sel, Q1, bq1.reshape(1, -1), Q2, bq2.reshape(1, -1))
    return out.reshape(1, K, DIM)
